# Initial kernel scaffold; baseline (speedup 1.0000x reference)
import jax
import jax.numpy as jnp
from jax.experimental import pallas as pl


def kernel(x, pe_index, pe_weight):
    raise NotImplementedError("write your pallas kernel here")


# speedup vs baseline: 1.2762x; 1.2762x over previous
import functools

import jax
import jax.numpy as jnp
from jax import lax
from jax.experimental import pallas as pl
from jax.experimental.pallas import tpu as pltpu
from jax.experimental.pallas import tpu_sc as plsc

D = 768
N = 4 * 8192
NC, NS = 2, 16
NW = NC * NS
PER_W = N // NW
CHUNK = 64


def _sc_gather(idx_flat, table):
    mesh = plsc.VectorSubcoreMesh(core_axis_name="c", subcore_axis_name="s")

    @functools.partial(
        pl.kernel,
        out_type=jax.ShapeDtypeStruct((N, D), jnp.float32),
        mesh=mesh,
        scratch_types=[
            pltpu.VMEM((CHUNK,), jnp.int32),
            pltpu.VMEM((CHUNK, D), jnp.float32),
            pltpu.SemaphoreType.DMA,
        ],
    )
    def gather_kernel(idx_hbm, table_hbm, out_hbm, idx_v, rows_v, sem):
        wid = lax.axis_index("s") * NC + lax.axis_index("c")
        base = wid * PER_W

        @pl.loop(0, PER_W, step=CHUNK)
        def _(off):
            pltpu.sync_copy(idx_hbm.at[pl.ds(base + off, CHUNK)], idx_v)
            pltpu.async_copy(table_hbm.at[idx_v], rows_v, sem).wait()
            pltpu.sync_copy(rows_v, out_hbm.at[pl.ds(base + off, CHUNK)])

    return gather_kernel(idx_flat, table)


def _tc_add(a, b):
    rb = 512

    def add_body(a_ref, b_ref, o_ref):
        o_ref[...] = a_ref[...] + b_ref[...]

    return pl.pallas_call(
        add_body,
        grid=(N // rb,),
        in_specs=[
            pl.BlockSpec((rb, D), lambda i: (i, 0)),
            pl.BlockSpec((rb, D), lambda i: (i, 0)),
        ],
        out_specs=pl.BlockSpec((rb, D), lambda i: (i, 0)),
        out_shape=jax.ShapeDtypeStruct((N, D), jnp.float32),
    )(a, b)


def kernel(x, pe_index, pe_weight):
    b, s, d = x.shape
    x_flat = x.reshape(N, D)
    idx_flat = pe_index.reshape(N).astype(jnp.int32)
    gathered = _sc_gather(idx_flat, pe_weight)
    out = _tc_add(x_flat, gathered)
    return out.reshape(b, s, d)


# speedup vs baseline: 2.0218x; 1.5842x over previous
import functools

import jax
import jax.numpy as jnp
from jax import lax
from jax.experimental import pallas as pl
from jax.experimental.pallas import tpu as pltpu
from jax.experimental.pallas import tpu_sc as plsc

D = 768
N = 4 * 8192
NC, NS = 2, 16
NW = NC * NS
PER_W = N // NW
C = 32
NCH = PER_W // C


def _sc_fused(idx3d, x2d, table):
    mesh = plsc.VectorSubcoreMesh(core_axis_name="c", subcore_axis_name="s")

    @functools.partial(
        pl.kernel,
        out_type=jax.ShapeDtypeStruct((N, D), jnp.float32),
        mesh=mesh,
        scratch_types=[
            pltpu.VMEM((NCH, C), jnp.int32),
            pltpu.VMEM((2, C, D), jnp.float32),
            pltpu.VMEM((2, C, D), jnp.float32),
            pltpu.SemaphoreType.DMA((2,)),
            pltpu.SemaphoreType.DMA((2,)),
            pltpu.SemaphoreType.DMA((2,)),
        ],
    )
    def fused_kernel(idx_hbm, x_hbm, table_hbm, out_hbm,
                     idx_v, xb, rb, semx, semg, semo):
        wid = lax.axis_index("s") * NC + lax.axis_index("c")
        base = wid * PER_W
        pltpu.sync_copy(idx_hbm.at[wid], idx_v)

        def rows(t):
            return pl.ds(base + t * C, C)

        def start_in(t, p):
            pltpu.async_copy(x_hbm.at[rows(t)], xb.at[p], semx.at[p])
            pltpu.async_copy(table_hbm.at[idx_v.at[t]], rb.at[p], semg.at[p])

        def wait_in(t, p):
            pltpu.make_async_copy(x_hbm.at[rows(t)], xb.at[p], semx.at[p]).wait()
            pltpu.make_async_copy(
                table_hbm.at[idx_v.at[t]], rb.at[p], semg.at[p]).wait()

        def start_out(t, p):
            pltpu.async_copy(xb.at[p], out_hbm.at[rows(t)], semo.at[p])

        def wait_out(t, p):
            pltpu.make_async_copy(xb.at[p], out_hbm.at[rows(t)], semo.at[p]).wait()

        def add(p):
            @pl.loop(0, C)
            def _(r):
                @plsc.parallel_loop(0, D, step=16, unroll=4)
                def _(j):
                    sl = pl.ds(j, 16)
                    xb[p, r, sl] = xb[p, r, sl] + rb[p, r, sl]

        start_in(0, 0)

        @pl.loop(0, NCH, step=2)
        def _(t0):
            for p in (0, 1):
                t = t0 + p
                q = 1 - p

                @pl.when(t > 0)
                def _():
                    wait_out(t - 1, q)

                @pl.when(t + 1 < NCH)
                def _():
                    start_in(t + 1, q)

                wait_in(t, p)
                add(p)
                start_out(t, p)

        wait_out(NCH - 1, (NCH - 1) % 2)

    return fused_kernel(idx3d, x2d, table)


def kernel(x, pe_index, pe_weight):
    b, s, d = x.shape
    x2d = x.reshape(N, D)
    idx3d = pe_index.reshape(NW, NCH, C).astype(jnp.int32)
    out = _sc_fused(idx3d, x2d, pe_weight)
    return out.reshape(b, s, d)
